# baseline (speedup 1.0000x reference)
import jax
import jax.numpy as jnp
from jax import lax
from jax.experimental import pallas as pl
from jax.experimental.pallas import tpu as pltpu, tpu_sc as plsc

B, S, D = 4, 8192, 768
NC, NS = 2, 16
NW = NC * NS
ROWS_PER_W = S // NW
CH = 32
NCHUNK = ROWS_PER_W // CH
LANES = 16
NCOL = D // LANES
NSTEP = NCHUNK * B


NXBUF = 3


def _sc_body(x_hbm, pos_hbm, out_hbm, x_v0, x_v1, x_v2, pos_v0, pos_v1,
             ld0, ld1, ld2, st0, st1, st2, ps0, ps1):
    x_bufs = [x_v0, x_v1, x_v2]
    pos_bufs = [pos_v0, pos_v1]
    ld_sems = [ld0, ld1, ld2]
    st_sems = [st0, st1, st2]
    pos_sems = [ps0, ps1]

    wid = lax.axis_index("s") * NC + lax.axis_index("c")
    seq_row0 = wid * ROWS_PER_W

    def x_row(step):
        c, b = divmod(step, B)
        return b * S + seq_row0 + c * CH

    def start_x_load(step):
        return pltpu.async_copy(
            x_hbm.at[pl.ds(x_row(step), CH)], x_bufs[step % NXBUF],
            ld_sems[step % NXBUF])

    def start_pos_load(c):
        return pltpu.async_copy(
            pos_hbm.at[pl.ds(seq_row0 + c * CH, CH)], pos_bufs[c % 2],
            pos_sems[c % 2])

    ld_h = [None] * NSTEP
    st_h = [None] * NSTEP
    pos_h = [None] * NCHUNK

    pos_h[0] = start_pos_load(0)
    ld_h[0] = start_x_load(0)
    ld_h[1] = start_x_load(1)
    if NCHUNK > 1:
        pos_h[1] = start_pos_load(1)

    for s in range(NSTEP):
        c, b = divmod(s, B)
        if s + 2 < NSTEP:
            if s + 2 - NXBUF >= 0:
                for h_ in st_h[s + 2 - NXBUF]:
                    h_.wait()
            ld_h[s + 2] = start_x_load(s + 2)
        ld_h[s].wait()
        if b == 0:
            pos_h[c].wait()

        buf = x_bufs[s % NXBUF]
        pbuf = pos_bufs[c % 2]

        half_h = []
        for h in range(4):
            r0 = h * (CH // 4)

            @plsc.parallel_loop(0, (CH // 4) * NCOL, 1, unroll=8)
            def _(i):
                r = r0 + (i & (CH // 4 - 1))
                k = (i >> 3) * LANES
                buf[r, pl.ds(k, LANES)] = (
                    buf[r, pl.ds(k, LANES)] + pbuf[r, pl.ds(k, LANES)]
                )

            half_h.append(pltpu.async_copy(
                buf.at[pl.ds(r0, CH // 4)],
                out_hbm.at[pl.ds(x_row(s) + r0, CH // 4)],
                st_sems[s % NXBUF]))
        st_h[s] = half_h

        if b == B - 1 and c + 2 < NCHUNK:
            pos_h[c + 2] = start_pos_load(c + 2)

    for s in range(NSTEP - NXBUF, NSTEP):
        for h_ in st_h[s]:
            h_.wait()


@jax.jit
def kernel(x, pos_table):
    mesh = plsc.VectorSubcoreMesh(
        core_axis_name="c", subcore_axis_name="s", num_cores=NC, num_subcores=NS
    )
    sc_call = pl.kernel(
        _sc_body,
        out_type=jax.ShapeDtypeStruct((B * S, D), jnp.float32),
        mesh=mesh,
        scratch_types=(
            [pltpu.VMEM((CH, D), jnp.float32)] * (NXBUF + 2)
            + [pltpu.SemaphoreType.DMA] * (2 * NXBUF + 2)
        ),
    )
    out = sc_call(x.reshape(B * S, D), pos_table)
    return out.reshape(B, S, D)


# speedup vs baseline: 1.0447x; 1.0447x over previous
import jax
import jax.numpy as jnp
from jax import lax
from jax.experimental import pallas as pl
from jax.experimental.pallas import tpu as pltpu, tpu_sc as plsc

B, S, D = 4, 8192, 768
NC, NS = 2, 16
NW = NC * NS
ROWS_PER_W = S // NW
CH = 32
NCHUNK = ROWS_PER_W // CH
LANES = 16
NCOL = D // LANES
NSTEP = NCHUNK * B


NXBUF = 3


def _sc_body(x_hbm, pos_hbm, out_hbm, x_v0, x_v1, x_v2, pos_v0, pos_v1,
             ld0, ld1, ld2, st0, st1, st2, ps0, ps1):
    x_bufs = [x_v0, x_v1, x_v2]
    pos_bufs = [pos_v0, pos_v1]
    ld_sems = [ld0, ld1, ld2]
    st_sems = [st0, st1, st2]
    pos_sems = [ps0, ps1]

    wid = lax.axis_index("s") * NC + lax.axis_index("c")
    seq_row0 = wid * ROWS_PER_W

    def x_row(step):
        c, b = divmod(step, B)
        return b * S + seq_row0 + c * CH

    def start_x_load(step):
        return pltpu.async_copy(
            x_hbm.at[pl.ds(x_row(step), CH)], x_bufs[step % NXBUF],
            ld_sems[step % NXBUF])

    def start_pos_load(c):
        return pltpu.async_copy(
            pos_hbm.at[pl.ds(seq_row0 + c * CH, CH)], pos_bufs[c % 2],
            pos_sems[c % 2])

    ld_h = [None] * NSTEP
    st_h = [None] * NSTEP
    pos_h = [None] * NCHUNK

    pos_h[0] = start_pos_load(0)
    ld_h[0] = start_x_load(0)
    ld_h[1] = start_x_load(1)
    if NCHUNK > 1:
        pos_h[1] = start_pos_load(1)

    for s in range(NSTEP):
        c, b = divmod(s, B)
        if s + 2 < NSTEP:
            if s + 2 - NXBUF >= 0:
                for h_ in st_h[s + 2 - NXBUF]:
                    h_.wait()
            ld_h[s + 2] = start_x_load(s + 2)
        ld_h[s].wait()
        if b == 0:
            pos_h[c].wait()

        buf = x_bufs[s % NXBUF]
        pbuf = pos_bufs[c % 2]

        half_h = []
        for h in range(2):
            r0 = h * (CH // 2)

            @plsc.parallel_loop(0, (CH // 2) * NCOL, 1, unroll=8)
            def _(i):
                r = r0 + (i & (CH // 2 - 1))
                k = (i >> 4) * LANES
                buf[r, pl.ds(k, LANES)] = (
                    buf[r, pl.ds(k, LANES)] + pbuf[r, pl.ds(k, LANES)]
                )

            half_h.append(pltpu.async_copy(
                buf.at[pl.ds(r0, CH // 2)],
                out_hbm.at[pl.ds(x_row(s) + r0, CH // 2)],
                st_sems[s % NXBUF]))
        st_h[s] = half_h

        if b == B - 1 and c + 2 < NCHUNK:
            pos_h[c + 2] = start_pos_load(c + 2)

    for s in range(NSTEP - NXBUF, NSTEP):
        for h_ in st_h[s]:
            h_.wait()


@jax.jit
def kernel(x, pos_table):
    mesh = plsc.VectorSubcoreMesh(
        core_axis_name="c", subcore_axis_name="s", num_cores=NC, num_subcores=NS
    )
    sc_call = pl.kernel(
        _sc_body,
        out_type=jax.ShapeDtypeStruct((B * S, D), jnp.float32),
        mesh=mesh,
        scratch_types=(
            [pltpu.VMEM((CH, D), jnp.float32)] * (NXBUF + 2)
            + [pltpu.SemaphoreType.DMA] * (2 * NXBUF + 2)
        ),
    )
    out = sc_call(x.reshape(B * S, D), pos_table)
    return out.reshape(B, S, D)


# speedup vs baseline: 1.0448x; 1.0000x over previous
import jax
import jax.numpy as jnp
from jax import lax
from jax.experimental import pallas as pl
from jax.experimental.pallas import tpu as pltpu, tpu_sc as plsc

B, S, D = 4, 8192, 768
NC, NS = 2, 16
NW = NC * NS
ROWS_PER_W = S // NW
CH = 32
NCHUNK = ROWS_PER_W // CH
LANES = 16
NCOL = D // LANES
NSTEP = NCHUNK * B


NXBUF = 3


def _sc_body(x_hbm, pos_hbm, out_hbm, x_v0, x_v1, x_v2, pos_v0, pos_v1,
             ld0, ld1, ld2, st0, st1, st2, ps0, ps1):
    x_bufs = [x_v0, x_v1, x_v2]
    pos_bufs = [pos_v0, pos_v1]
    ld_sems = [ld0, ld1, ld2]
    st_sems = [st0, st1, st2]
    pos_sems = [ps0, ps1]

    wid = lax.axis_index("s") * NC + lax.axis_index("c")
    seq_row0 = wid * ROWS_PER_W

    def x_row(step):
        c, b = divmod(step, B)
        return b * S + seq_row0 + c * CH

    def start_x_load(step):
        return pltpu.async_copy(
            x_hbm.at[pl.ds(x_row(step), CH)], x_bufs[step % NXBUF],
            ld_sems[step % NXBUF])

    def start_pos_load(c):
        return pltpu.async_copy(
            pos_hbm.at[pl.ds(seq_row0 + c * CH, CH)], pos_bufs[c % 2],
            pos_sems[c % 2])

    ld_h = [None] * NSTEP
    st_h = [None] * NSTEP
    pos_h = [None] * NCHUNK

    pos_h[0] = start_pos_load(0)
    ld_h[0] = start_x_load(0)
    ld_h[1] = start_x_load(1)
    if NCHUNK > 1:
        pos_h[1] = start_pos_load(1)

    for s in range(NSTEP):
        c, b = divmod(s, B)
        ld_h[s].wait()
        if b == 0:
            pos_h[c].wait()

        buf = x_bufs[s % NXBUF]
        pbuf = pos_bufs[c % 2]

        half_h = []
        for h in range(2):
            r0 = h * (CH // 2)

            @plsc.parallel_loop(0, (CH // 2) * NCOL, 1, unroll=8)
            def _(i):
                r = r0 + (i & (CH // 2 - 1))
                k = (i >> 4) * LANES
                buf[r, pl.ds(k, LANES)] = (
                    buf[r, pl.ds(k, LANES)] + pbuf[r, pl.ds(k, LANES)]
                )

            half_h.append(pltpu.async_copy(
                buf.at[pl.ds(r0, CH // 2)],
                out_hbm.at[pl.ds(x_row(s) + r0, CH // 2)],
                st_sems[s % NXBUF]))
            if h == 0 and s + 2 < NSTEP:
                if s + 2 - NXBUF >= 0:
                    for h_ in st_h[s + 2 - NXBUF]:
                        h_.wait()
                ld_h[s + 2] = start_x_load(s + 2)
        st_h[s] = half_h

        if b == B - 1 and c + 2 < NCHUNK:
            pos_h[c + 2] = start_pos_load(c + 2)

    for s in range(NSTEP - NXBUF, NSTEP):
        for h_ in st_h[s]:
            h_.wait()


@jax.jit
def kernel(x, pos_table):
    mesh = plsc.VectorSubcoreMesh(
        core_axis_name="c", subcore_axis_name="s", num_cores=NC, num_subcores=NS
    )
    sc_call = pl.kernel(
        _sc_body,
        out_type=jax.ShapeDtypeStruct((B * S, D), jnp.float32),
        mesh=mesh,
        scratch_types=(
            [pltpu.VMEM((CH, D), jnp.float32)] * (NXBUF + 2)
            + [pltpu.SemaphoreType.DMA] * (2 * NXBUF + 2)
        ),
    )
    out = sc_call(x.reshape(B * S, D), pos_table)
    return out.reshape(B, S, D)
